# baseline (speedup 1.0000x reference)
import functools

import jax
import jax.numpy as jnp
from jax import lax
from jax.experimental import pallas as pl
from jax.experimental.pallas import tpu as pltpu
from jax.experimental.pallas import tpu_sc as plsc

F = 26
D = 32
VOCAB = 100000
NC, NS, L = 2, 16, 16
NW = NC * NS
C = 128


def _embedding_bag(idx_t, table):
    B = idx_t.shape[1]
    rows_per_w = B // NW
    n_chunks = rows_per_w // C

    mesh = plsc.VectorSubcoreMesh(core_axis_name="c", subcore_axis_name="s")

    @functools.partial(
        pl.kernel,
        out_type=jax.ShapeDtypeStruct((B, D), jnp.float32),
        mesh=mesh,
        compiler_params=pltpu.CompilerParams(use_tc_tiling_on_sc=False),
        scratch_types=[
            pltpu.VMEM((F, C), jnp.int32),
            pltpu.VMEM((F, C, D), jnp.float32),
            pltpu.VMEM((C, D), jnp.float32),
            pltpu.SemaphoreType.DMA,
        ],
    )
    def body(idx_hbm, tab_hbm, out_hbm, idx_v, rows_v, out_v, sem):
        wid = lax.axis_index("s") * NC + lax.axis_index("c")
        base = wid * rows_per_w

        def chunk(j, carry):
            cbase = base + j * C
            for f in range(F):
                pltpu.sync_copy(idx_hbm.at[f, pl.ds(cbase, C)], idx_v.at[f])

            def offs(i, _):
                for f in range(F):
                    sl = pl.ds(i * L, L)
                    idx_v[f, sl] = idx_v[f, sl] + f * VOCAB
                return 0

            lax.fori_loop(0, C // L, offs, 0)

            copies = [
                pltpu.async_copy(tab_hbm.at[idx_v.at[f]], rows_v.at[f], sem)
                for f in range(F)
            ]
            for cp in copies:
                cp.wait()

            def red(c, _):
                for h in range(D // L):
                    sl = pl.ds(h * L, L)
                    acc = rows_v[0, c, sl]
                    for f in range(1, F):
                        acc = acc + rows_v[f, c, sl]
                    out_v[c, sl] = acc
                return 0

            lax.fori_loop(0, C, red, 0)
            pltpu.sync_copy(out_v, out_hbm.at[pl.ds(cbase, C)])
            return carry

        lax.fori_loop(0, n_chunks, chunk, 0)

    return body(idx_t, table)


def kernel(input_, embed_weight):
    idx_t = input_.T
    return _embedding_bag(idx_t, embed_weight)


# speedup vs baseline: 1.0100x; 1.0100x over previous
import functools

import jax
import jax.numpy as jnp
import numpy as np
from jax import lax
from jax.experimental import pallas as pl
from jax.experimental.pallas import tpu as pltpu
from jax.experimental.pallas import tpu_sc as plsc

F = 26
D = 32
VOCAB = 100000
NC, NS, L = 2, 16, 16
NW = NC * NS
C = 128
K = C * F
G = 128
NG = K // G

_OFFS = ((np.arange(K, dtype=np.int32) % F) * VOCAB).reshape(NG, G)


def _embedding_bag(idx2d, offs2d, table):
    B = idx2d.shape[0] * G // F
    rows_per_w = B // NW
    n_chunks = rows_per_w // C
    idxrows_per_chunk = K // G

    mesh = plsc.VectorSubcoreMesh(core_axis_name="c", subcore_axis_name="s")

    @functools.partial(
        pl.kernel,
        out_type=jax.ShapeDtypeStruct((B, D), jnp.float32),
        mesh=mesh,
        compiler_params=pltpu.CompilerParams(use_tc_tiling_on_sc=False),
        scratch_types=[
            pltpu.VMEM((NG, G), jnp.int32),
            pltpu.VMEM((NG, G), jnp.int32),
            pltpu.VMEM((K, D), jnp.float32),
            pltpu.VMEM((C, D), jnp.float32),
            pltpu.SemaphoreType.DMA,
        ],
    )
    def body(idx_hbm, offs_hbm, tab_hbm, out_hbm, idx_v, offs_v, rows_v, out_v, sem):
        wid = lax.axis_index("s") * NC + lax.axis_index("c")
        bag_base = wid * rows_per_w
        pltpu.sync_copy(offs_hbm, offs_v)

        def chunk(j, carry):
            cbag = bag_base + j * C
            crow = cbag * F // G
            pltpu.sync_copy(idx_hbm.at[pl.ds(crow, idxrows_per_chunk)], idx_v)

            def offs(i, _):
                for g in range(NG):
                    sl = pl.ds(i * L, L)
                    idx_v[g, sl] = idx_v[g, sl] + offs_v[g, sl]
                return 0

            lax.fori_loop(0, G // L, offs, 0)

            copies = [
                pltpu.async_copy(
                    tab_hbm.at[idx_v.at[g]], rows_v.at[pl.ds(g * G, G)], sem
                )
                for g in range(NG)
            ]
            for cp in copies:
                cp.wait()

            def red(c, _):
                rbase = c * F
                for h in range(D // L):
                    sl = pl.ds(h * L, L)
                    acc = rows_v[rbase, sl]
                    for f in range(1, F):
                        acc = acc + rows_v[rbase + f, sl]
                    out_v[c, sl] = acc
                return 0

            lax.fori_loop(0, C, red, 0)
            pltpu.sync_copy(out_v, out_hbm.at[pl.ds(cbag, C)])
            return carry

        lax.fori_loop(0, n_chunks, chunk, 0)

    return body(idx2d, offs2d, table)


def kernel(input_, embed_weight):
    B = input_.shape[0]
    idx2d = input_.reshape(B * F // G, G)
    return _embedding_bag(idx2d, jnp.asarray(_OFFS), embed_weight)


# speedup vs baseline: 1.1733x; 1.1617x over previous
import functools

import jax
import jax.numpy as jnp
import numpy as np
from jax import lax
from jax.experimental import pallas as pl
from jax.experimental.pallas import tpu as pltpu
from jax.experimental.pallas import tpu_sc as plsc

F = 26
D = 32
VOCAB = 100000
NC, NS, L = 2, 16, 16
NW = NC * NS
C = 128
K = C * F
G = 128
NG = K // G

TBLK = 2048

_OFFS = ((np.arange(K, dtype=np.int32) % F) * VOCAB).reshape(NG, G)


def _tc_relayout(tab_t):
    V = tab_t.shape[1]
    n_blk = pl.cdiv(V, TBLK)
    Q = TBLK // 4

    def body(in_ref, out_ref):
        xt = in_ref[...].T
        out_ref[...] = jnp.concatenate(
            [xt[j * Q:(j + 1) * Q, :] for j in range(4)], axis=1
        )

    return pl.pallas_call(
        body,
        grid=(n_blk,),
        in_specs=[pl.BlockSpec((D, TBLK), lambda i: (0, i))],
        out_specs=pl.BlockSpec((Q, 128), lambda i: (i, 0)),
        out_shape=jax.ShapeDtypeStruct((n_blk * Q, 128), jnp.float32),
    )(tab_t)


def _embedding_bag(idx2d, offs2d, table):
    B = idx2d.shape[0] * G // F
    rows_per_w = B // NW
    n_chunks = rows_per_w // C
    idxrows_per_chunk = K // G

    mesh = plsc.VectorSubcoreMesh(core_axis_name="c", subcore_axis_name="s")

    @functools.partial(
        pl.kernel,
        out_type=jax.ShapeDtypeStruct((B, D), jnp.float32),
        mesh=mesh,
        compiler_params=pltpu.CompilerParams(use_tc_tiling_on_sc=False),
        scratch_types=[
            pltpu.VMEM((NG, G), jnp.int32),
            pltpu.VMEM((NG, G), jnp.int32),
            pltpu.VMEM((K, D), jnp.float32),
            pltpu.VMEM((C, D), jnp.float32),
            pltpu.SemaphoreType.DMA,
        ],
    )
    def body(idx_hbm, offs_hbm, tab_hbm, out_hbm, idx_v, offs_v, rows_v, out_v, sem):
        wid = lax.axis_index("s") * NC + lax.axis_index("c")
        bag_base = wid * rows_per_w
        pltpu.sync_copy(offs_hbm, offs_v)

        def chunk(j, carry):
            cbag = bag_base + j * C
            crow = cbag * F // G
            pltpu.sync_copy(idx_hbm.at[pl.ds(crow, idxrows_per_chunk)], idx_v)

            def offs(i, _):
                for g in range(NG):
                    sl = pl.ds(i * L, L)
                    r = idx_v[g, sl] + offs_v[g, sl]
                    band = jnp.bitwise_and(r, TBLK - 1)
                    s = (r - band) + jnp.left_shift(
                        jnp.bitwise_and(r, TBLK // 4 - 1), 2
                    ) + jnp.right_shift(band, 9)
                    idx_v[g, sl] = s
                return 0

            lax.fori_loop(0, G // L, offs, 0)

            copies = [
                pltpu.async_copy(
                    tab_hbm.at[idx_v.at[g]], rows_v.at[pl.ds(g * G, G)], sem
                )
                for g in range(NG)
            ]
            for cp in copies:
                cp.wait()

            def red(c, _):
                rbase = c * F
                for h in range(D // L):
                    sl = pl.ds(h * L, L)
                    acc = rows_v[rbase, sl]
                    for f in range(1, F):
                        acc = acc + rows_v[rbase + f, sl]
                    out_v[c, sl] = acc
                return 0

            lax.fori_loop(0, C, red, 0)
            pltpu.sync_copy(out_v, out_hbm.at[pl.ds(cbag, C)])
            return carry

        lax.fori_loop(0, n_chunks, chunk, 0)

    return body(idx2d, offs2d, table)


def kernel(input_, embed_weight):
    B = input_.shape[0]
    idx2d = input_.reshape(B * F // G, G)
    tab_t = embed_weight.T
    tab128 = _tc_relayout(tab_t)
    tab_lin = tab128.reshape(-1, D)
    return _embedding_bag(idx2d, jnp.asarray(_OFFS), tab_lin)


# speedup vs baseline: 2.2772x; 1.9408x over previous
import functools

import jax
import jax.numpy as jnp
import numpy as np
from jax import lax
from jax.experimental import pallas as pl
from jax.experimental.pallas import tpu as pltpu
from jax.experimental.pallas import tpu_sc as plsc

F = 26
D = 32
VOCAB = 100000
NC, NS, L = 2, 16, 16
NW = NC * NS
C = 128
K = C * F
G = 128
NG = K // G

TBLK = 4096
QSH = (TBLK // 4).bit_length() - 1

_OFFS = ((np.arange(K, dtype=np.int32) % F) * VOCAB).reshape(NG, G)


def _tc_relayout(tab_t):
    V = tab_t.shape[1]
    n_blk = pl.cdiv(V, TBLK)
    Q = TBLK // 4

    def body(in_ref, out_ref):
        x = in_ref[...]
        y = jnp.concatenate(
            [x[:, j * Q:(j + 1) * Q] for j in range(4)], axis=0
        )
        out_ref[...] = y.T

    return pl.pallas_call(
        body,
        grid=(n_blk,),
        in_specs=[pl.BlockSpec((D, TBLK), lambda i: (0, i))],
        out_specs=pl.BlockSpec((Q, 128), lambda i: (i, 0)),
        out_shape=jax.ShapeDtypeStruct((n_blk * Q, 128), jnp.float32),
    )(tab_t)


def _embedding_bag(idx2d, offs2d, table):
    B = idx2d.shape[0] * G // F
    rows_per_w = B // NW
    n_chunks = rows_per_w // C
    idxrows_per_chunk = K // G

    mesh = plsc.VectorSubcoreMesh(core_axis_name="c", subcore_axis_name="s")

    @functools.partial(
        pl.kernel,
        out_type=jax.ShapeDtypeStruct((B, D), jnp.float32),
        mesh=mesh,
        compiler_params=pltpu.CompilerParams(use_tc_tiling_on_sc=False),
        scratch_types=[
            pltpu.VMEM((NG, G), jnp.int32),
            pltpu.VMEM((NG, G), jnp.int32),
            pltpu.VMEM((K, D), jnp.float32),
            pltpu.VMEM((C, D), jnp.float32),
            pltpu.SemaphoreType.DMA,
        ],
    )
    def body(idx_hbm, offs_hbm, tab_hbm, out_hbm, idx_v, offs_v, rows_v, out_v, sem):
        wid = lax.axis_index("s") * NC + lax.axis_index("c")
        bag_base = wid * rows_per_w
        pltpu.sync_copy(offs_hbm, offs_v)

        def chunk(j, carry):
            cbag = bag_base + j * C
            crow = cbag * F // G
            pltpu.sync_copy(idx_hbm.at[pl.ds(crow, idxrows_per_chunk)], idx_v)

            def offs(i, _):
                for g in range(NG):
                    sl = pl.ds(i * L, L)
                    r = idx_v[g, sl] + offs_v[g, sl]
                    band = jnp.bitwise_and(r, TBLK - 1)
                    s = (r - band) + jnp.left_shift(
                        jnp.bitwise_and(r, TBLK // 4 - 1), 2
                    ) + jnp.right_shift(band, QSH)
                    idx_v[g, sl] = s
                return 0

            lax.fori_loop(0, G // L, offs, 0)

            copies = [
                pltpu.async_copy(
                    tab_hbm.at[idx_v.at[g]], rows_v.at[pl.ds(g * G, G)], sem
                )
                for g in range(NG)
            ]
            for cp in copies:
                cp.wait()

            def red(c, _):
                rbase = c * F
                for h in range(D // L):
                    sl = pl.ds(h * L, L)
                    acc = rows_v[rbase, sl]
                    for f in range(1, F):
                        acc = acc + rows_v[rbase + f, sl]
                    out_v[c, sl] = acc
                return 0

            lax.fori_loop(0, C, red, 0)
            pltpu.sync_copy(out_v, out_hbm.at[pl.ds(cbag, C)])
            return carry

        lax.fori_loop(0, n_chunks, chunk, 0)

    return body(idx2d, offs2d, table)


def kernel(input_, embed_weight):
    B = input_.shape[0]
    idx2d = input_.reshape(B * F // G, G)
    tab_t = embed_weight.T
    tab128 = _tc_relayout(tab_t)
    tab_lin = tab128.reshape(-1, D)
    return _embedding_bag(idx2d, jnp.asarray(_OFFS), tab_lin)


# speedup vs baseline: 3.0182x; 1.3254x over previous
import functools

import jax
import jax.numpy as jnp
import numpy as np
from jax import lax
from jax.experimental import pallas as pl
from jax.experimental.pallas import tpu as pltpu
from jax.experimental.pallas import tpu_sc as plsc

F = 26
D = 32
VOCAB = 100000
NC, NS, L = 2, 16, 16
NW = NC * NS
C = 128
K = C * F
G = 128
NG = K // G

TBLK = 8192
QSH = (TBLK // 4).bit_length() - 1

_OFFS = ((np.arange(K, dtype=np.int32) % F) * VOCAB).reshape(NG, G)


def _tc_relayout(tab_t):
    V = tab_t.shape[1]
    n_blk = pl.cdiv(V, TBLK)
    Q = TBLK // 4

    def body(in_ref, out_ref):
        x = in_ref[...]
        y = jnp.concatenate(
            [x[:, j * Q:(j + 1) * Q] for j in range(4)], axis=0
        )
        out_ref[...] = y.T

    return pl.pallas_call(
        body,
        grid=(n_blk,),
        in_specs=[pl.BlockSpec((D, TBLK), lambda i: (0, i))],
        out_specs=pl.BlockSpec((Q, 128), lambda i: (i, 0)),
        out_shape=jax.ShapeDtypeStruct((n_blk * Q, 128), jnp.float32),
    )(tab_t)


def _embedding_bag(idx2d, offs2d, table):
    B = idx2d.shape[0] * G // F
    rows_per_w = B // NW
    n_chunks = rows_per_w // C
    idxrows_per_chunk = K // G

    mesh = plsc.VectorSubcoreMesh(core_axis_name="c", subcore_axis_name="s")

    @functools.partial(
        pl.kernel,
        out_type=jax.ShapeDtypeStruct((B, D), jnp.float32),
        mesh=mesh,
        compiler_params=pltpu.CompilerParams(use_tc_tiling_on_sc=False),
        scratch_types=[
            pltpu.VMEM((NG, G), jnp.int32),
            pltpu.VMEM((NG, G), jnp.int32),
            pltpu.VMEM((K, D), jnp.float32),
            pltpu.VMEM((C, D), jnp.float32),
            pltpu.SemaphoreType.DMA,
        ],
    )
    def body(idx_hbm, offs_hbm, tab_hbm, out_hbm, idx_v, offs_v, rows_v, out_v, sem):
        wid = lax.axis_index("s") * NC + lax.axis_index("c")
        bag_base = wid * rows_per_w
        pltpu.sync_copy(offs_hbm, offs_v)

        def chunk(j, carry):
            cbag = bag_base + j * C
            crow = cbag * F // G
            pltpu.sync_copy(idx_hbm.at[pl.ds(crow, idxrows_per_chunk)], idx_v)

            def offs(i, _):
                for g in range(NG):
                    sl = pl.ds(i * L, L)
                    r = idx_v[g, sl] + offs_v[g, sl]
                    band = jnp.bitwise_and(r, TBLK - 1)
                    s = (r - band) + jnp.left_shift(
                        jnp.bitwise_and(r, TBLK // 4 - 1), 2
                    ) + jnp.right_shift(band, QSH)
                    idx_v[g, sl] = s
                return 0

            lax.fori_loop(0, G // L, offs, 0)

            copies = [
                pltpu.async_copy(
                    tab_hbm.at[idx_v.at[g]], rows_v.at[pl.ds(g * G, G)], sem
                )
                for g in range(NG)
            ]
            for cp in copies:
                cp.wait()

            def red(c, _):
                rbase = c * F
                for h in range(D // L):
                    sl = pl.ds(h * L, L)
                    acc = rows_v[rbase, sl]
                    for f in range(1, F):
                        acc = acc + rows_v[rbase + f, sl]
                    out_v[c, sl] = acc
                return 0

            lax.fori_loop(0, C, red, 0)
            pltpu.sync_copy(out_v, out_hbm.at[pl.ds(cbag, C)])
            return carry

        lax.fori_loop(0, n_chunks, chunk, 0)

    return body(idx2d, offs2d, table)


def kernel(input_, embed_weight):
    B = input_.shape[0]
    idx2d = input_.reshape(B * F // G, G)
    tab_t = embed_weight.T
    tab128 = _tc_relayout(tab_t)
    tab_lin = tab128.reshape(-1, D)
    return _embedding_bag(idx2d, jnp.asarray(_OFFS), tab_lin)


# speedup vs baseline: 3.8324x; 1.2698x over previous
import functools

import jax
import jax.numpy as jnp
import numpy as np
from jax import lax
from jax.experimental import pallas as pl
from jax.experimental.pallas import tpu as pltpu
from jax.experimental.pallas import tpu_sc as plsc

F = 26
D = 32
VOCAB = 100000
NC, NS, L = 2, 16, 16
NW = NC * NS
C = 128
K = C * F
G = 128
NG = K // G

TBLK = 16384
QSH = (TBLK // 4).bit_length() - 1

_OFFS = ((np.arange(K, dtype=np.int32) % F) * VOCAB).reshape(NG, G)


def _tc_relayout(tab_t):
    V = tab_t.shape[1]
    n_blk = pl.cdiv(V, TBLK)
    Q = TBLK // 4

    def body(in_ref, out_ref):
        x = in_ref[...]
        y = jnp.concatenate(
            [x[:, j * Q:(j + 1) * Q] for j in range(4)], axis=0
        )
        out_ref[...] = y.T

    return pl.pallas_call(
        body,
        grid=(n_blk,),
        in_specs=[pl.BlockSpec((D, TBLK), lambda i: (0, i))],
        out_specs=pl.BlockSpec((Q, 128), lambda i: (i, 0)),
        out_shape=jax.ShapeDtypeStruct((n_blk * Q, 128), jnp.float32),
    )(tab_t)


def _embedding_bag(idx2d, offs2d, table):
    B = idx2d.shape[0] * G // F
    rows_per_w = B // NW
    n_chunks = rows_per_w // C
    idxrows_per_chunk = K // G

    mesh = plsc.VectorSubcoreMesh(core_axis_name="c", subcore_axis_name="s")

    @functools.partial(
        pl.kernel,
        out_type=jax.ShapeDtypeStruct((B, D), jnp.float32),
        mesh=mesh,
        compiler_params=pltpu.CompilerParams(use_tc_tiling_on_sc=False),
        scratch_types=[
            pltpu.VMEM((NG, G), jnp.int32),
            pltpu.VMEM((NG, G), jnp.int32),
            pltpu.VMEM((K, D), jnp.float32),
            pltpu.VMEM((C, D), jnp.float32),
            pltpu.SemaphoreType.DMA,
        ],
    )
    def body(idx_hbm, offs_hbm, tab_hbm, out_hbm, idx_v, offs_v, rows_v, out_v, sem):
        wid = lax.axis_index("s") * NC + lax.axis_index("c")
        bag_base = wid * rows_per_w
        pltpu.sync_copy(offs_hbm, offs_v)

        def chunk(j, carry):
            cbag = bag_base + j * C
            crow = cbag * F // G
            pltpu.sync_copy(idx_hbm.at[pl.ds(crow, idxrows_per_chunk)], idx_v)

            def offs(i, _):
                for g in range(NG):
                    sl = pl.ds(i * L, L)
                    r = idx_v[g, sl] + offs_v[g, sl]
                    band = jnp.bitwise_and(r, TBLK - 1)
                    s = (r - band) + jnp.left_shift(
                        jnp.bitwise_and(r, TBLK // 4 - 1), 2
                    ) + jnp.right_shift(band, QSH)
                    idx_v[g, sl] = s
                return 0

            lax.fori_loop(0, G // L, offs, 0)

            copies = [
                pltpu.async_copy(
                    tab_hbm.at[idx_v.at[g]], rows_v.at[pl.ds(g * G, G)], sem
                )
                for g in range(NG)
            ]
            for cp in copies:
                cp.wait()

            def red(c, _):
                rbase = c * F
                for h in range(D // L):
                    sl = pl.ds(h * L, L)
                    acc = rows_v[rbase, sl]
                    for f in range(1, F):
                        acc = acc + rows_v[rbase + f, sl]
                    out_v[c, sl] = acc
                return 0

            lax.fori_loop(0, C, red, 0)
            pltpu.sync_copy(out_v, out_hbm.at[pl.ds(cbag, C)])
            return carry

        lax.fori_loop(0, n_chunks, chunk, 0)

    return body(idx2d, offs2d, table)


def kernel(input_, embed_weight):
    B = input_.shape[0]
    idx2d = input_.reshape(B * F // G, G)
    tab_t = embed_weight.T
    tab128 = _tc_relayout(tab_t)
    tab_lin = tab128.reshape(-1, D)
    return _embedding_bag(idx2d, jnp.asarray(_OFFS), tab_lin)


# speedup vs baseline: 4.2721x; 1.1147x over previous
import functools

import jax
import jax.numpy as jnp
import numpy as np
from jax import lax
from jax.experimental import pallas as pl
from jax.experimental.pallas import tpu as pltpu
from jax.experimental.pallas import tpu_sc as plsc

F = 26
D = 32
VOCAB = 100000
NC, NS, L = 2, 16, 16
NW = NC * NS
C = 128
K = C * F
G = 128
NG = K // G

TBLK = 32768
QSH = (TBLK // 4).bit_length() - 1

_OFFS = ((np.arange(K, dtype=np.int32) % F) * VOCAB).reshape(NG, G)


def _tc_relayout(tab_t):
    V = tab_t.shape[1]
    n_blk = pl.cdiv(V, TBLK)
    Q = TBLK // 4

    def body(in_ref, out_ref):
        x = in_ref[...]
        y = jnp.concatenate(
            [x[:, j * Q:(j + 1) * Q] for j in range(4)], axis=0
        )
        out_ref[...] = y.T

    return pl.pallas_call(
        body,
        grid=(n_blk,),
        in_specs=[pl.BlockSpec((D, TBLK), lambda i: (0, i))],
        out_specs=pl.BlockSpec((Q, 128), lambda i: (i, 0)),
        out_shape=jax.ShapeDtypeStruct((n_blk * Q, 128), jnp.float32),
    )(tab_t)


def _embedding_bag(idx2d, offs2d, table):
    B = idx2d.shape[0] * G // F
    rows_per_w = B // NW
    n_chunks = rows_per_w // C
    idxrows_per_chunk = K // G

    mesh = plsc.VectorSubcoreMesh(core_axis_name="c", subcore_axis_name="s")

    @functools.partial(
        pl.kernel,
        out_type=jax.ShapeDtypeStruct((B, D), jnp.float32),
        mesh=mesh,
        compiler_params=pltpu.CompilerParams(use_tc_tiling_on_sc=False),
        scratch_types=[
            pltpu.VMEM((NG, G), jnp.int32),
            pltpu.VMEM((NG, G), jnp.int32),
            pltpu.VMEM((K, D), jnp.float32),
            pltpu.VMEM((C, D), jnp.float32),
            pltpu.SemaphoreType.DMA,
        ],
    )
    def body(idx_hbm, offs_hbm, tab_hbm, out_hbm, idx_v, offs_v, rows_v, out_v, sem):
        wid = lax.axis_index("s") * NC + lax.axis_index("c")
        bag_base = wid * rows_per_w
        pltpu.sync_copy(offs_hbm, offs_v)

        def chunk(j, carry):
            cbag = bag_base + j * C
            crow = cbag * F // G
            pltpu.sync_copy(idx_hbm.at[pl.ds(crow, idxrows_per_chunk)], idx_v)

            def offs(i, _):
                for g in range(NG):
                    sl = pl.ds(i * L, L)
                    r = idx_v[g, sl] + offs_v[g, sl]
                    band = jnp.bitwise_and(r, TBLK - 1)
                    s = (r - band) + jnp.left_shift(
                        jnp.bitwise_and(r, TBLK // 4 - 1), 2
                    ) + jnp.right_shift(band, QSH)
                    idx_v[g, sl] = s
                return 0

            lax.fori_loop(0, G // L, offs, 0)

            copies = [
                pltpu.async_copy(
                    tab_hbm.at[idx_v.at[g]], rows_v.at[pl.ds(g * G, G)], sem
                )
                for g in range(NG)
            ]
            for cp in copies:
                cp.wait()

            def red(c, _):
                rbase = c * F
                for h in range(D // L):
                    sl = pl.ds(h * L, L)
                    acc = rows_v[rbase, sl]
                    for f in range(1, F):
                        acc = acc + rows_v[rbase + f, sl]
                    out_v[c, sl] = acc
                return 0

            lax.fori_loop(0, C, red, 0)
            pltpu.sync_copy(out_v, out_hbm.at[pl.ds(cbag, C)])
            return carry

        lax.fori_loop(0, n_chunks, chunk, 0)

    return body(idx2d, offs2d, table)


def kernel(input_, embed_weight):
    B = input_.shape[0]
    idx2d = input_.reshape(B * F // G, G)
    tab_t = embed_weight.T
    tab128 = _tc_relayout(tab_t)
    tab_lin = tab128.reshape(-1, D)
    return _embedding_bag(idx2d, jnp.asarray(_OFFS), tab_lin)


# speedup vs baseline: 4.3273x; 1.0129x over previous
import functools

import jax
import jax.numpy as jnp
import numpy as np
from jax import lax
from jax.experimental import pallas as pl
from jax.experimental.pallas import tpu as pltpu
from jax.experimental.pallas import tpu_sc as plsc

F = 26
D = 32
VOCAB = 100000
NC, NS, L = 2, 16, 16
NW = NC * NS
C = 128
K = C * F
G = 128
NG = K // G

TBLK = 65536
QSH = (TBLK // 4).bit_length() - 1

_OFFS = ((np.arange(K, dtype=np.int32) % F) * VOCAB).reshape(NG, G)


def _tc_relayout(tab_t):
    V = tab_t.shape[1]
    n_blk = pl.cdiv(V, TBLK)
    Q = TBLK // 4

    def body(in_ref, out_ref):
        x = in_ref[...]
        y = jnp.concatenate(
            [x[:, j * Q:(j + 1) * Q] for j in range(4)], axis=0
        )
        out_ref[...] = y.T

    return pl.pallas_call(
        body,
        grid=(n_blk,),
        in_specs=[pl.BlockSpec((D, TBLK), lambda i: (0, i))],
        out_specs=pl.BlockSpec((Q, 128), lambda i: (i, 0)),
        out_shape=jax.ShapeDtypeStruct((n_blk * Q, 128), jnp.float32),
    )(tab_t)


def _embedding_bag(idx2d, offs2d, table):
    B = idx2d.shape[0] * G // F
    rows_per_w = B // NW
    n_chunks = rows_per_w // C
    idxrows_per_chunk = K // G

    mesh = plsc.VectorSubcoreMesh(core_axis_name="c", subcore_axis_name="s")

    @functools.partial(
        pl.kernel,
        out_type=jax.ShapeDtypeStruct((B, D), jnp.float32),
        mesh=mesh,
        compiler_params=pltpu.CompilerParams(use_tc_tiling_on_sc=False),
        scratch_types=[
            pltpu.VMEM((NG, G), jnp.int32),
            pltpu.VMEM((NG, G), jnp.int32),
            pltpu.VMEM((K, D), jnp.float32),
            pltpu.VMEM((C, D), jnp.float32),
            pltpu.SemaphoreType.DMA,
        ],
    )
    def body(idx_hbm, offs_hbm, tab_hbm, out_hbm, idx_v, offs_v, rows_v, out_v, sem):
        wid = lax.axis_index("s") * NC + lax.axis_index("c")
        bag_base = wid * rows_per_w
        pltpu.sync_copy(offs_hbm, offs_v)

        def chunk(j, carry):
            cbag = bag_base + j * C
            crow = cbag * F // G
            pltpu.sync_copy(idx_hbm.at[pl.ds(crow, idxrows_per_chunk)], idx_v)

            def offs(i, _):
                for g in range(NG):
                    sl = pl.ds(i * L, L)
                    r = idx_v[g, sl] + offs_v[g, sl]
                    band = jnp.bitwise_and(r, TBLK - 1)
                    s = (r - band) + jnp.left_shift(
                        jnp.bitwise_and(r, TBLK // 4 - 1), 2
                    ) + jnp.right_shift(band, QSH)
                    idx_v[g, sl] = s
                return 0

            lax.fori_loop(0, G // L, offs, 0)

            copies = [
                pltpu.async_copy(
                    tab_hbm.at[idx_v.at[g]], rows_v.at[pl.ds(g * G, G)], sem
                )
                for g in range(NG)
            ]
            for cp in copies:
                cp.wait()

            def red(c, _):
                rbase = c * F
                for h in range(D // L):
                    sl = pl.ds(h * L, L)
                    acc = rows_v[rbase, sl]
                    for f in range(1, F):
                        acc = acc + rows_v[rbase + f, sl]
                    out_v[c, sl] = acc
                return 0

            lax.fori_loop(0, C, red, 0)
            pltpu.sync_copy(out_v, out_hbm.at[pl.ds(cbag, C)])
            return carry

        lax.fori_loop(0, n_chunks, chunk, 0)

    return body(idx2d, offs2d, table)


def kernel(input_, embed_weight):
    B = input_.shape[0]
    idx2d = input_.reshape(B * F // G, G)
    tab_t = embed_weight.T
    tab128 = _tc_relayout(tab_t)
    tab_lin = tab128.reshape(-1, D)
    return _embedding_bag(idx2d, jnp.asarray(_OFFS), tab_lin)
